# baseline (speedup 1.0000x reference)
import jax
import jax.numpy as jnp
from jax import lax
from jax.experimental import pallas as pl
from jax.experimental.pallas import tpu as pltpu

_NUM_NODES = 1000000
_EMBED_DIM = 64
_BC = 8192
_GRID = (_NUM_NODES + _BC - 1) // _BC


def _copy_body(w_ref, o_ref):
    o_ref[...] = w_ref[...]


def kernel(weight):
    wt = weight.T
    out_t = pl.pallas_call(
        _copy_body,
        out_shape=jax.ShapeDtypeStruct((_EMBED_DIM, _NUM_NODES), jnp.float32),
        grid=(_GRID,),
        in_specs=[pl.BlockSpec((_EMBED_DIM, _BC), lambda i: (0, i))],
        out_specs=pl.BlockSpec((_EMBED_DIM, _BC), lambda i: (0, i)),
    )(wt)
    return out_t.T
